# Initial kernel scaffold; baseline (speedup 1.0000x reference)
import jax
import jax.numpy as jnp
from jax.experimental import pallas as pl


def kernel(obs, adj_matrix, W1, b1, a1, W2, b2, a2):
    raise NotImplementedError("write your pallas kernel here")


# speedup vs baseline: 8.3363x; 8.3363x over previous
import functools

import jax
import jax.numpy as jnp
from jax import lax
from jax.experimental import pallas as pl
from jax.experimental.pallas import tpu as pltpu

N = 1024
H = 4
NEG = -9e15


def _gat_layer_body(R, C, fin, fout, elu_out, last_softmax,
                    x_ref, adj_ref, W_ref, b_ref, a_ref, out_ref,
                    feats_ref, slc_ref, srr_ref):
    i = pl.program_id(0)

    @pl.when(i == 0)
    def _prologue():
        x = x_ref[...]
        W = W_ref[...]
        feats = lax.dot_general(x, W, (((1,), (1,)), ((), ())),
                                preferred_element_type=jnp.float32)
        feats = feats + b_ref[...][None, :]
        feats_ref[...] = feats
        a = a_ref[...]
        for h in range(H):
            fh = feats[:, h * C:(h + 1) * C]
            sl = jnp.sum(fh * a[h, :C][None, :], axis=1, keepdims=True)
            sr = jnp.sum(fh * a[h, C:][None, :], axis=1, keepdims=True)
            slc_ref[:, h:h + 1] = sl
            srr_ref[h:h + 1, :] = jnp.transpose(sr)

    adjb = adj_ref[...]
    maskb = adjb != 0
    outs = []
    for h in range(H):
        slh = slc_ref[pl.ds(i * R, R), h:h + 1]
        srh = srr_ref[h:h + 1, :]
        logit = slh + srh
        leaky = jnp.maximum(logit, 0.2 * logit)
        masked = jnp.where(maskb, leaky, NEG)
        m = jnp.max(masked, axis=1, keepdims=True)
        e = jnp.exp(masked - m)
        s = jnp.sum(e, axis=1, keepdims=True)
        fh = feats_ref[:, h * C:(h + 1) * C]
        oh = jnp.dot(e, fh, preferred_element_type=jnp.float32) / s
        outs.append(oh)
    res = jnp.concatenate(outs, axis=1)
    if elu_out:
        res = jnp.where(res > 0, res, jnp.exp(res) - 1.0)
    if last_softmax:
        mm = jnp.max(res, axis=1, keepdims=True)
        ee = jnp.exp(res - mm)
        res = ee / jnp.sum(ee, axis=1, keepdims=True)
    out_ref[...] = res


def _gat_layer(x, adj, W, b, a, C, elu_out, last_softmax, R=128):
    fin = x.shape[1]
    fout = H * C
    body = functools.partial(_gat_layer_body, R, C, fin, fout,
                             elu_out, last_softmax)
    return pl.pallas_call(
        body,
        grid=(N // R,),
        in_specs=[
            pl.BlockSpec((N, fin), lambda i: (0, 0)),
            pl.BlockSpec((R, N), lambda i: (i, 0)),
            pl.BlockSpec((fout, fin), lambda i: (0, 0)),
            pl.BlockSpec((fout,), lambda i: (0,)),
            pl.BlockSpec((H, 2 * C), lambda i: (0, 0)),
        ],
        out_specs=pl.BlockSpec((R, fout), lambda i: (i, 0)),
        out_shape=jax.ShapeDtypeStruct((N, fout), jnp.float32),
        scratch_shapes=[
            pltpu.VMEM((N, fout), jnp.float32),
            pltpu.VMEM((N, H), jnp.float32),
            pltpu.VMEM((H, N), jnp.float32),
        ],
        compiler_params=pltpu.CompilerParams(
            dimension_semantics=("arbitrary",),
        ),
    )(x, adj, W, b, a)


@jax.jit
def kernel(obs, adj_matrix, W1, b1, a1, W2, b2, a2):
    x = obs.reshape(N, -1)
    adj = adj_matrix.reshape(N, N)
    h1 = _gat_layer(x, adj, W1, b1, a1, C=16, elu_out=True, last_softmax=False)
    out = _gat_layer(h1, adj, W2, b2, a2, C=8, elu_out=False, last_softmax=True)
    return out


# speedup vs baseline: 8.5536x; 1.0261x over previous
import functools

import jax
import jax.numpy as jnp
from jax import lax
from jax.experimental import pallas as pl
from jax.experimental.pallas import tpu as pltpu

N = 1024
H = 4


def _gat_layer_body(R, C, fin, fout, elu_out, last_softmax,
                    x_ref, adj_ref, W_ref, b_ref, Al_ref, Ar_ref, out_ref,
                    feats_ref, slc_ref, srr_ref, msr_ref, cmean_ref):
    i = pl.program_id(0)

    @pl.when(i == 0)
    def _prologue():
        x = x_ref[...]
        W = W_ref[...]
        feats = lax.dot_general(x, W, (((1,), (1,)), ((), ())),
                                preferred_element_type=jnp.float32)
        feats = feats + b_ref[...][None, :]
        feats_ref[...] = feats
        slc_ref[...] = jnp.dot(feats, Al_ref[...],
                               preferred_element_type=jnp.float32)
        sr = jnp.dot(feats, Ar_ref[...],
                     preferred_element_type=jnp.float32)
        srr = jnp.transpose(sr)
        srr_ref[...] = srr
        msr_ref[...] = jnp.max(srr, axis=1, keepdims=True)
        cmean_ref[...] = jnp.mean(feats, axis=0, keepdims=True)

    adjb = adj_ref[...]
    maskf = (adjb != 0).astype(jnp.float32)
    outs = []
    for h in range(H):
        slh = slc_ref[pl.ds(i * R, R), h:h + 1]
        srh = srr_ref[h:h + 1, :]
        msrh = msr_ref[h:h + 1, 0:1]
        mlin = slh + msrh
        mih = jnp.maximum(mlin, 0.2 * mlin)
        logit = slh + srh
        leaky = jnp.maximum(logit, 0.2 * logit)
        e = jnp.exp(leaky - mih) * maskf
        s = jnp.sum(e, axis=1, keepdims=True)
        fh = feats_ref[:, h * C:(h + 1) * C]
        num = jnp.dot(e, fh, preferred_element_type=jnp.float32)
        s_safe = jnp.where(s > 0, s, 1.0)
        cm = cmean_ref[0:1, h * C:(h + 1) * C]
        oh = jnp.where(s > 0, num / s_safe, cm)
        outs.append(oh)
    res = jnp.concatenate(outs, axis=1)
    if elu_out:
        res = jnp.where(res > 0, res, jnp.exp(res) - 1.0)
    if last_softmax:
        mm = jnp.max(res, axis=1, keepdims=True)
        ee = jnp.exp(res - mm)
        res = ee / jnp.sum(ee, axis=1, keepdims=True)
    out_ref[...] = res


def _gat_layer(x, adj, W, b, a, C, elu_out, last_softmax, R=128):
    fin = x.shape[1]
    fout = H * C
    eye = jnp.eye(H, dtype=a.dtype)
    Al = (a[:, :C, None] * eye[:, None, :]).reshape(fout, H)
    Ar = (a[:, C:, None] * eye[:, None, :]).reshape(fout, H)
    body = functools.partial(_gat_layer_body, R, C, fin, fout,
                             elu_out, last_softmax)
    return pl.pallas_call(
        body,
        grid=(N // R,),
        in_specs=[
            pl.BlockSpec((N, fin), lambda i: (0, 0)),
            pl.BlockSpec((R, N), lambda i: (i, 0)),
            pl.BlockSpec((fout, fin), lambda i: (0, 0)),
            pl.BlockSpec((fout,), lambda i: (0,)),
            pl.BlockSpec((fout, H), lambda i: (0, 0)),
            pl.BlockSpec((fout, H), lambda i: (0, 0)),
        ],
        out_specs=pl.BlockSpec((R, fout), lambda i: (i, 0)),
        out_shape=jax.ShapeDtypeStruct((N, fout), jnp.float32),
        scratch_shapes=[
            pltpu.VMEM((N, fout), jnp.float32),
            pltpu.VMEM((N, H), jnp.float32),
            pltpu.VMEM((H, N), jnp.float32),
            pltpu.VMEM((H, 1), jnp.float32),
            pltpu.VMEM((1, fout), jnp.float32),
        ],
        compiler_params=pltpu.CompilerParams(
            dimension_semantics=("arbitrary",),
        ),
    )(x, adj, W, b, Al, Ar)


@jax.jit
def kernel(obs, adj_matrix, W1, b1, a1, W2, b2, a2):
    x = obs.reshape(N, -1)
    adj = adj_matrix.reshape(N, N)
    h1 = _gat_layer(x, adj, W1, b1, a1, C=16, elu_out=True, last_softmax=False)
    out = _gat_layer(h1, adj, W2, b2, a2, C=8, elu_out=False, last_softmax=True)
    return out


# speedup vs baseline: 11.4585x; 1.3396x over previous
import functools

import jax
import jax.numpy as jnp
from jax import lax
from jax.experimental import pallas as pl
from jax.experimental.pallas import tpu as pltpu

N = 1024
H = 4


def _gat_layer_body(R, C, fin, fout, elu_out, last_softmax,
                    x_ref, adj_ref, W_ref, b_ref, A_ref, out_ref,
                    u_ref, p_ref, v_ref, q_ref, fbd_ref, cmean_ref):
    i = pl.program_id(0)

    @pl.when(i == 0)
    def _prologue():
        x = x_ref[...]
        W = W_ref[...]
        feats = lax.dot_general(x, W, (((1,), (1,)), ((), ())),
                                preferred_element_type=jnp.float32)
        feats = feats + b_ref[...][None, :]
        sall = jnp.dot(feats, A_ref[...],
                       preferred_element_type=jnp.float32)
        sl = sall[:, :H]
        sr = sall[:, H:]
        msr = jnp.max(sr, axis=0, keepdims=True)
        xm = sl + msr
        m = jnp.maximum(xm, 0.2 * xm)
        u_ref[...] = jnp.exp(xm - m)
        p_ref[...] = jnp.exp(0.2 * xm - m)
        srm = sr - msr
        v_ref[...] = jnp.transpose(jnp.exp(srm))
        q_ref[...] = jnp.transpose(jnp.exp(0.2 * srm))
        cmean_ref[...] = jnp.mean(feats, axis=0, keepdims=True)
        blocks = []
        for h in range(H):
            hm = ((lax.broadcasted_iota(jnp.int32, (1, fout), 1) // C) == h)
            fb_h = feats * hm.astype(jnp.float32)
            oh = (lax.broadcasted_iota(jnp.int32, (1, H), 1) == h)
            ones_h = jnp.broadcast_to(oh.astype(jnp.float32), (N, H))
            blocks.append(jnp.concatenate([fb_h, ones_h], axis=1))
        fbd_ref[...] = jnp.concatenate(blocks, axis=0)

    adjb = adj_ref[...]
    maskf = (adjb != 0).astype(jnp.float32)
    es = []
    for h in range(H):
        uh = u_ref[pl.ds(i * R, R), h:h + 1]
        ph = p_ref[pl.ds(i * R, R), h:h + 1]
        vh = v_ref[h:h + 1, :]
        qh = q_ref[h:h + 1, :]
        es.append(jnp.maximum(uh * vh, ph * qh) * maskf)
    E = jnp.concatenate(es, axis=1)
    O = jnp.dot(E, fbd_ref[...],
                preferred_element_type=jnp.float32)
    outs = []
    for h in range(H):
        s = O[:, fout + h:fout + h + 1]
        num = O[:, h * C:(h + 1) * C]
        cm = cmean_ref[0:1, h * C:(h + 1) * C]
        s_safe = jnp.where(s > 0, s, 1.0)
        outs.append(jnp.where(s > 0, num / s_safe, cm))
    res = jnp.concatenate(outs, axis=1)
    if elu_out:
        res = jnp.where(res > 0, res, jnp.exp(res) - 1.0)
    if last_softmax:
        mm = jnp.max(res, axis=1, keepdims=True)
        ee = jnp.exp(res - mm)
        res = ee / jnp.sum(ee, axis=1, keepdims=True)
    out_ref[...] = res


def _gat_layer(x, adj, W, b, a, C, elu_out, last_softmax, R=128):
    fin = x.shape[1]
    fout = H * C
    eye = jnp.eye(H, dtype=a.dtype)
    Al = (a[:, :C, None] * eye[:, None, :]).reshape(fout, H)
    Ar = (a[:, C:, None] * eye[:, None, :]).reshape(fout, H)
    A = jnp.concatenate([Al, Ar], axis=1)
    body = functools.partial(_gat_layer_body, R, C, fin, fout,
                             elu_out, last_softmax)
    return pl.pallas_call(
        body,
        grid=(N // R,),
        in_specs=[
            pl.BlockSpec((N, fin), lambda i: (0, 0)),
            pl.BlockSpec((R, N), lambda i: (i, 0)),
            pl.BlockSpec((fout, fin), lambda i: (0, 0)),
            pl.BlockSpec((fout,), lambda i: (0,)),
            pl.BlockSpec((fout, 2 * H), lambda i: (0, 0)),
        ],
        out_specs=pl.BlockSpec((R, fout), lambda i: (i, 0)),
        out_shape=jax.ShapeDtypeStruct((N, fout), jnp.float32),
        scratch_shapes=[
            pltpu.VMEM((N, H), jnp.float32),
            pltpu.VMEM((N, H), jnp.float32),
            pltpu.VMEM((H, N), jnp.float32),
            pltpu.VMEM((H, N), jnp.float32),
            pltpu.VMEM((H * N, fout + H), jnp.float32),
            pltpu.VMEM((1, fout), jnp.float32),
        ],
        compiler_params=pltpu.CompilerParams(
            dimension_semantics=("arbitrary",),
        ),
    )(x, adj, W, b, A)


@jax.jit
def kernel(obs, adj_matrix, W1, b1, a1, W2, b2, a2):
    x = obs.reshape(N, -1)
    adj = adj_matrix.reshape(N, N)
    h1 = _gat_layer(x, adj, W1, b1, a1, C=16, elu_out=True, last_softmax=False)
    out = _gat_layer(h1, adj, W2, b2, a2, C=8, elu_out=False, last_softmax=True)
    return out


# speedup vs baseline: 11.9187x; 1.0402x over previous
import functools

import jax
import jax.numpy as jnp
from jax import lax
from jax.experimental import pallas as pl
from jax.experimental.pallas import tpu as pltpu

N = 1024
H = 4


def _gat_layer_body(R, C, fin, fout, elu_out, last_softmax,
                    x_ref, adj_ref, W_ref, b_ref, A_ref, out_ref,
                    up_ref, v_ref, q_ref, fbd_ref, cmean_ref):
    i = pl.program_id(0)

    @pl.when(i == 0)
    def _prologue():
        x = x_ref[...]
        W = W_ref[...]
        feats = lax.dot_general(x, W, (((1,), (1,)), ((), ())),
                                preferred_element_type=jnp.float32)
        feats = feats + b_ref[...][None, :]
        sall = jnp.dot(feats, A_ref[...],
                       preferred_element_type=jnp.float32)
        sallT = jnp.transpose(sall)
        slT = sallT[:H, :]
        srT = sallT[H:, :]
        msr = jnp.max(srT, axis=1, keepdims=True)
        xm = slT + msr
        m = jnp.maximum(xm, 0.2 * xm)
        upT = jnp.exp(jnp.concatenate([xm, 0.2 * xm], axis=0)
                      - jnp.concatenate([m, m], axis=0))
        up_ref[...] = jnp.transpose(upT)
        srm = srT - msr
        v_ref[...] = jnp.exp(srm)
        q_ref[...] = jnp.exp(0.2 * srm)
        cmean_ref[...] = jnp.dot(
            jnp.full((1, N), 1.0 / N, dtype=jnp.float32), feats,
            preferred_element_type=jnp.float32)
        blocks = []
        for h in range(H):
            hm = ((lax.broadcasted_iota(jnp.int32, (1, fout), 1) // C) == h)
            fb_h = feats * hm.astype(jnp.float32)
            oh = (lax.broadcasted_iota(jnp.int32, (1, H), 1) == h)
            ones_h = jnp.broadcast_to(oh.astype(jnp.float32), (N, H))
            blocks.append(jnp.concatenate([fb_h, ones_h], axis=1))
        fbd_ref[...] = jnp.concatenate(blocks, axis=0)

    adjb = adj_ref[...]
    maskf = (adjb != 0).astype(jnp.float32)
    O = jnp.zeros((R, fout + H), dtype=jnp.float32)
    for h in range(H):
        uh = up_ref[pl.ds(i * R, R), h:h + 1]
        ph = up_ref[pl.ds(i * R, R), H + h:H + h + 1]
        vh = v_ref[h:h + 1, :]
        qh = q_ref[h:h + 1, :]
        eh = jnp.maximum(uh * vh, ph * qh) * maskf
        O = O + jnp.dot(eh, fbd_ref[pl.ds(h * N, N), :],
                        preferred_element_type=jnp.float32)
    outs = []
    for h in range(H):
        s = O[:, fout + h:fout + h + 1]
        num = O[:, h * C:(h + 1) * C]
        cm = cmean_ref[0:1, h * C:(h + 1) * C]
        s_safe = jnp.where(s > 0, s, 1.0)
        outs.append(jnp.where(s > 0, num / s_safe, cm))
    res = jnp.concatenate(outs, axis=1)
    if elu_out:
        res = jnp.where(res > 0, res, jnp.exp(res) - 1.0)
    if last_softmax:
        mm = jnp.max(res, axis=1, keepdims=True)
        ee = jnp.exp(res - mm)
        res = ee / jnp.sum(ee, axis=1, keepdims=True)
    out_ref[...] = res


def _gat_layer(x, adj, W, b, a, C, elu_out, last_softmax, R=128):
    fin = x.shape[1]
    fout = H * C
    eye = jnp.eye(H, dtype=a.dtype)
    Al = (a[:, :C, None] * eye[:, None, :]).reshape(fout, H)
    Ar = (a[:, C:, None] * eye[:, None, :]).reshape(fout, H)
    A = jnp.concatenate([Al, Ar], axis=1)
    body = functools.partial(_gat_layer_body, R, C, fin, fout,
                             elu_out, last_softmax)
    return pl.pallas_call(
        body,
        grid=(N // R,),
        in_specs=[
            pl.BlockSpec((N, fin), lambda i: (0, 0)),
            pl.BlockSpec((R, N), lambda i: (i, 0)),
            pl.BlockSpec((fout, fin), lambda i: (0, 0)),
            pl.BlockSpec((fout,), lambda i: (0,)),
            pl.BlockSpec((fout, 2 * H), lambda i: (0, 0)),
        ],
        out_specs=pl.BlockSpec((R, fout), lambda i: (i, 0)),
        out_shape=jax.ShapeDtypeStruct((N, fout), jnp.float32),
        scratch_shapes=[
            pltpu.VMEM((N, 2 * H), jnp.float32),
            pltpu.VMEM((H, N), jnp.float32),
            pltpu.VMEM((H, N), jnp.float32),
            pltpu.VMEM((H * N, fout + H), jnp.float32),
            pltpu.VMEM((1, fout), jnp.float32),
        ],
        compiler_params=pltpu.CompilerParams(
            dimension_semantics=("arbitrary",),
        ),
    )(x, adj, W, b, A)


@jax.jit
def kernel(obs, adj_matrix, W1, b1, a1, W2, b2, a2):
    x = obs.reshape(N, -1)
    adj = adj_matrix.reshape(N, N)
    h1 = _gat_layer(x, adj, W1, b1, a1, C=16, elu_out=True, last_softmax=False)
    out = _gat_layer(h1, adj, W2, b2, a2, C=8, elu_out=False, last_softmax=True)
    return out


# speedup vs baseline: 14.7144x; 1.2346x over previous
import functools

import jax
import jax.numpy as jnp
from jax import lax
from jax.experimental import pallas as pl
from jax.experimental.pallas import tpu as pltpu

N = 1024
H = 4
R = 256
KB = N // R
C1, C2 = 16, 8
F1, F2 = H * C1, H * C2


def _prologue_compute(feats, A, C, fout,
                      up_ref, v_ref, q_ref, fbd_ref, cm_ref):
    sall = jnp.dot(feats, A, preferred_element_type=jnp.float32)
    sallT = jnp.transpose(sall)
    slT = sallT[:H, :]
    srT = sallT[H:, :]
    msr = jnp.max(srT, axis=1, keepdims=True)
    xm = slT + msr
    m = jnp.maximum(xm, 0.2 * xm)
    upT = jnp.exp(jnp.concatenate([xm, 0.2 * xm], axis=0)
                  - jnp.concatenate([m, m], axis=0))
    up_ref[...] = jnp.transpose(upT)
    srm = srT - msr
    v_ref[...] = jnp.exp(srm)
    q_ref[...] = jnp.exp(0.2 * srm)
    cm_ref[...] = jnp.dot(jnp.full((1, N), 1.0 / N, dtype=jnp.float32),
                          feats, preferred_element_type=jnp.float32)
    blocks = []
    for h in range(H):
        hm = ((lax.broadcasted_iota(jnp.int32, (1, fout), 1) // C) == h)
        fb_h = feats * hm.astype(jnp.float32)
        oh = (lax.broadcasted_iota(jnp.int32, (1, H), 1) == h)
        ones_h = jnp.broadcast_to(oh.astype(jnp.float32), (N, H))
        blocks.append(jnp.concatenate([fb_h, ones_h], axis=1))
    fbd_ref[...] = jnp.concatenate(blocks, axis=0)


def _attn_rows(i0, maskf, C, fout, up_ref, v_ref, q_ref, fbd_ref, cm_ref):
    O = jnp.zeros((R, fout + H), dtype=jnp.float32)
    for h in range(H):
        uh = up_ref[pl.ds(i0, R), h:h + 1]
        ph = up_ref[pl.ds(i0, R), H + h:H + h + 1]
        vh = v_ref[h:h + 1, :]
        qh = q_ref[h:h + 1, :]
        eh = jnp.maximum(uh * vh, ph * qh) * maskf
        O = O + jnp.dot(eh, fbd_ref[pl.ds(h * N, N), :],
                        preferred_element_type=jnp.float32)
    outs = []
    for h in range(H):
        s = O[:, fout + h:fout + h + 1]
        num = O[:, h * C:(h + 1) * C]
        cm = cm_ref[0:1, h * C:(h + 1) * C]
        s_safe = jnp.where(s > 0, s, 1.0)
        outs.append(jnp.where(s > 0, num / s_safe, cm))
    return jnp.concatenate(outs, axis=1)


def _fused_body(x_ref, adj_ref, W1_ref, b1_ref, A1_ref,
                W2_ref, b2_ref, A2_ref, out_ref,
                up1_ref, v1_ref, q1_ref, fbd1_ref, cm1_ref,
                up2_ref, v2_ref, q2_ref, fbd2_ref, cm2_ref,
                mask_ref, h1_ref):
    i = pl.program_id(0)

    @pl.when(i == 0)
    def _prologue1():
        x = x_ref[...]
        feats = lax.dot_general(x, W1_ref[...], (((1,), (1,)), ((), ())),
                                preferred_element_type=jnp.float32)
        feats = feats + b1_ref[...][None, :]
        _prologue_compute(feats, A1_ref[...], C1, F1,
                          up1_ref, v1_ref, q1_ref, fbd1_ref, cm1_ref)

    @pl.when(i == KB)
    def _prologue2():
        h1 = h1_ref[...]
        feats = lax.dot_general(h1, W2_ref[...], (((1,), (1,)), ((), ())),
                                preferred_element_type=jnp.float32)
        feats = feats + b2_ref[...][None, :]
        _prologue_compute(feats, A2_ref[...], C2, F2,
                          up2_ref, v2_ref, q2_ref, fbd2_ref, cm2_ref)

    @pl.when(i < KB)
    def _layer1_step():
        adjb = adj_ref[...]
        maskf = (adjb != 0).astype(jnp.float32)
        mask_ref[pl.ds(i * R, R), :] = maskf
        res = _attn_rows(i * R, maskf, C1, F1,
                         up1_ref, v1_ref, q1_ref, fbd1_ref, cm1_ref)
        res = jnp.where(res > 0, res, jnp.exp(res) - 1.0)
        h1_ref[pl.ds(i * R, R), :] = res

    @pl.when(i >= KB)
    def _layer2_step():
        i0 = (i - KB) * R
        maskf = mask_ref[pl.ds(i0, R), :]
        res = _attn_rows(i0, maskf, C2, F2,
                         up2_ref, v2_ref, q2_ref, fbd2_ref, cm2_ref)
        mm = jnp.max(res, axis=1, keepdims=True)
        ee = jnp.exp(res - mm)
        out_ref[pl.ds(i0, R), :] = ee / jnp.sum(ee, axis=1, keepdims=True)


def _head_proj(a, C):
    fout = H * C
    eye = jnp.eye(H, dtype=a.dtype)
    Al = (a[:, :C, None] * eye[:, None, :]).reshape(fout, H)
    Ar = (a[:, C:, None] * eye[:, None, :]).reshape(fout, H)
    return jnp.concatenate([Al, Ar], axis=1)


@jax.jit
def kernel(obs, adj_matrix, W1, b1, a1, W2, b2, a2):
    x = obs.reshape(N, -1)
    adj = adj_matrix.reshape(N, N)
    fin = x.shape[1]
    A1 = _head_proj(a1, C1)
    A2 = _head_proj(a2, C2)
    return pl.pallas_call(
        _fused_body,
        grid=(2 * KB,),
        in_specs=[
            pl.BlockSpec((N, fin), lambda i: (0, 0)),
            pl.BlockSpec((R, N), lambda i: (jnp.where(i < KB, i, 0), 0)),
            pl.BlockSpec((F1, fin), lambda i: (0, 0)),
            pl.BlockSpec((F1,), lambda i: (0,)),
            pl.BlockSpec((F1, 2 * H), lambda i: (0, 0)),
            pl.BlockSpec((F2, F1), lambda i: (0, 0)),
            pl.BlockSpec((F2,), lambda i: (0,)),
            pl.BlockSpec((F2, 2 * H), lambda i: (0, 0)),
        ],
        out_specs=pl.BlockSpec((N, F2), lambda i: (0, 0)),
        out_shape=jax.ShapeDtypeStruct((N, F2), jnp.float32),
        scratch_shapes=[
            pltpu.VMEM((N, 2 * H), jnp.float32),
            pltpu.VMEM((H, N), jnp.float32),
            pltpu.VMEM((H, N), jnp.float32),
            pltpu.VMEM((H * N, F1 + H), jnp.float32),
            pltpu.VMEM((1, F1), jnp.float32),
            pltpu.VMEM((N, 2 * H), jnp.float32),
            pltpu.VMEM((H, N), jnp.float32),
            pltpu.VMEM((H, N), jnp.float32),
            pltpu.VMEM((H * N, F2 + H), jnp.float32),
            pltpu.VMEM((1, F2), jnp.float32),
            pltpu.VMEM((N, N), jnp.float32),
            pltpu.VMEM((N, F1), jnp.float32),
        ],
        compiler_params=pltpu.CompilerParams(
            dimension_semantics=("arbitrary",),
        ),
    )(x, adj, W1, b1, A1, W2, b2, A2)
